# Initial kernel scaffold; baseline (speedup 1.0000x reference)
import jax
import jax.numpy as jnp
from jax.experimental import pallas as pl


def kernel(input_ids, weight, position_embedding):
    raise NotImplementedError("write your pallas kernel here")


# speedup vs baseline: 1.2625x; 1.2625x over previous
import functools

import jax
import jax.numpy as jnp
from jax import lax
from jax.experimental import pallas as pl
from jax.experimental.pallas import tpu as pltpu
from jax.experimental.pallas import tpu_sc as plsc

VOCAB = 49408
EMBED = 256
SEQ = 77
BATCH = 4096
NUM_WORKERS = 32
IDS_PER_W = BATCH * SEQ // NUM_WORKERS
CHUNK = 128
CHUNKS_PER_W = IDS_PER_W // CHUNK
POS_EXT = 208
LANES = 16
VREGS_PER_ROW = EMBED // LANES


@jax.jit
def _sc_embed(ids3, weight, position_embedding):
    mesh = plsc.VectorSubcoreMesh(core_axis_name="c", subcore_axis_name="s")

    @functools.partial(
        pl.kernel,
        out_type=jax.ShapeDtypeStruct((BATCH * SEQ, EMBED), jnp.float32),
        mesh=mesh,
        scratch_types=[
            pltpu.VMEM((CHUNKS_PER_W, CHUNK), jnp.int32),
            pltpu.VMEM((POS_EXT, EMBED), jnp.float32),
            pltpu.VMEM((CHUNK, EMBED), jnp.float32),
            pltpu.SemaphoreType.DMA,
        ],
    )
    def k(ids_hbm, w_hbm, pos_hbm, out_hbm, idx_v, pos_v, rows_v, sem):
        wid = lax.axis_index("s") * 2 + lax.axis_index("c")
        base = wid * IDS_PER_W
        pltpu.sync_copy(ids_hbm.at[wid], idx_v)
        pltpu.sync_copy(pos_hbm, pos_v)

        def chunk_body(c, carry):
            pltpu.async_copy(w_hbm.at[idx_v.at[c]], rows_v, sem).wait()
            q = lax.rem(c * CHUNK, SEQ)

            def add_body(r, _):
                p = q + r
                for j in range(VREGS_PER_ROW):
                    sl = pl.ds(j * LANES, LANES)
                    rows_v[r, sl] = rows_v[r, sl] + pos_v[p, sl]
                return _

            lax.fori_loop(0, CHUNK, add_body, 0)
            pltpu.sync_copy(rows_v, out_hbm.at[pl.ds(base + c * CHUNK, CHUNK)])
            return carry

        lax.fori_loop(0, CHUNKS_PER_W, chunk_body, 0)

    return k(ids3, weight, position_embedding)


def kernel(input_ids, weight, position_embedding):
    ids3 = jnp.asarray(input_ids, jnp.int32).reshape(
        NUM_WORKERS, CHUNKS_PER_W, CHUNK)
    pos_ext = jnp.concatenate(
        [position_embedding, position_embedding,
         position_embedding[: POS_EXT - 2 * SEQ]], axis=0)
    out = _sc_embed(ids3, weight, pos_ext)
    return out.reshape(BATCH, SEQ, EMBED)


# speedup vs baseline: 1.4096x; 1.1165x over previous
import functools

import jax
import jax.numpy as jnp
from jax import lax
from jax.experimental import pallas as pl
from jax.experimental.pallas import tpu as pltpu
from jax.experimental.pallas import tpu_sc as plsc

VOCAB = 49408
EMBED = 256
SEQ = 77
BATCH = 4096
NUM_WORKERS = 32
IDS_PER_W = BATCH * SEQ // NUM_WORKERS
CHUNK = 128
CHUNKS_PER_W = IDS_PER_W // CHUNK
POS_EXT = 208
LANES = 16
VREGS_PER_ROW = EMBED // LANES


@jax.jit
def _sc_embed(ids3, weight, position_embedding):
    mesh = plsc.VectorSubcoreMesh(core_axis_name="c", subcore_axis_name="s")

    @functools.partial(
        pl.kernel,
        out_type=jax.ShapeDtypeStruct((BATCH * SEQ, EMBED), jnp.float32),
        mesh=mesh,
        scratch_types=[
            pltpu.VMEM((CHUNKS_PER_W, CHUNK), jnp.int32),
            pltpu.VMEM((POS_EXT, EMBED), jnp.float32),
            pltpu.VMEM((CHUNK, EMBED), jnp.float32),
            pltpu.VMEM((CHUNK, EMBED), jnp.float32),
            pltpu.SemaphoreType.DMA,
            pltpu.SemaphoreType.DMA,
        ],
    )
    def k(ids_hbm, w_hbm, pos_hbm, out_hbm, idx_v, pos_v,
          rows_a, rows_b, sem_a, sem_b):
        wid = lax.axis_index("s") * 2 + lax.axis_index("c")
        base = wid * IDS_PER_W
        pltpu.sync_copy(ids_hbm.at[wid], idx_v)
        pltpu.sync_copy(pos_hbm, pos_v)

        def gather(c, buf, sem):
            return pltpu.async_copy(w_hbm.at[idx_v.at[c]], buf, sem)

        def add_and_flush(c, buf):
            q = lax.rem(c * CHUNK, SEQ)

            def add_body(r, _):
                p = q + r
                for j in range(VREGS_PER_ROW):
                    sl = pl.ds(j * LANES, LANES)
                    buf[r, sl] = buf[r, sl] + pos_v[p, sl]
                return _

            lax.fori_loop(0, CHUNK, add_body, 0)
            pltpu.sync_copy(buf, out_hbm.at[pl.ds(base + c * CHUNK, CHUNK)])

        gather(0, rows_a, sem_a)

        def pair_body(i, carry):
            c0 = 2 * i
            pltpu.make_async_copy(w_hbm.at[idx_v.at[c0]], rows_a, sem_a).wait()
            gather(c0 + 1, rows_b, sem_b)
            add_and_flush(c0, rows_a)
            pltpu.make_async_copy(w_hbm.at[idx_v.at[c0 + 1]], rows_b, sem_b).wait()
            gather(c0 + 2, rows_a, sem_a)
            add_and_flush(c0 + 1, rows_b)
            return carry

        lax.fori_loop(0, (CHUNKS_PER_W - 1) // 2, pair_body, 0)
        pltpu.make_async_copy(
            w_hbm.at[idx_v.at[CHUNKS_PER_W - 1]], rows_a, sem_a).wait()
        add_and_flush(CHUNKS_PER_W - 1, rows_a)

    return k(ids3, weight, position_embedding)


def kernel(input_ids, weight, position_embedding):
    ids3 = jnp.asarray(input_ids, jnp.int32).reshape(
        NUM_WORKERS, CHUNKS_PER_W, CHUNK)
    pos_ext = jnp.concatenate(
        [position_embedding, position_embedding,
         position_embedding[: POS_EXT - 2 * SEQ]], axis=0)
    out = _sc_embed(ids3, weight, pos_ext)
    return out.reshape(BATCH, SEQ, EMBED)


# speedup vs baseline: 2.3137x; 1.6414x over previous
import functools

import jax
import jax.numpy as jnp
from jax import lax
from jax.experimental import pallas as pl
from jax.experimental.pallas import tpu as pltpu
from jax.experimental.pallas import tpu_sc as plsc

VOCAB = 49408
EMBED = 256
SEQ = 77
BATCH = 4096
NUM_WORKERS = 32
IDS_PER_W = BATCH * SEQ // NUM_WORKERS
CHUNK = 128
CHUNKS_PER_W = IDS_PER_W // CHUNK
POS_EXT = 208
LANES = 16
VREGS_PER_ROW = EMBED // LANES


@jax.jit
def _sc_embed(ids3, weight, position_embedding):
    mesh = plsc.VectorSubcoreMesh(core_axis_name="c", subcore_axis_name="s")

    @functools.partial(
        pl.kernel,
        out_type=jax.ShapeDtypeStruct((BATCH * SEQ, EMBED), jnp.float32),
        mesh=mesh,
        scratch_types=[
            pltpu.VMEM((CHUNKS_PER_W, CHUNK), jnp.int32),
            pltpu.VMEM((POS_EXT, EMBED), jnp.float32),
            pltpu.VMEM((CHUNK, EMBED), jnp.float32),
            pltpu.VMEM((CHUNK, EMBED), jnp.float32),
            pltpu.SemaphoreType.DMA,
            pltpu.SemaphoreType.DMA,
        ],
    )
    def k(ids_hbm, w_hbm, pos_hbm, out_hbm, idx_v, pos_v,
          rows_a, rows_b, sem_a, sem_b):
        wid = lax.axis_index("s") * 2 + lax.axis_index("c")
        base = wid * IDS_PER_W
        pltpu.sync_copy(ids_hbm.at[wid], idx_v)
        pltpu.sync_copy(pos_hbm, pos_v)

        def gather(c, buf, sem):
            return pltpu.async_copy(w_hbm.at[idx_v.at[c]], buf, sem)

        def add_and_flush(c, buf):
            q = lax.rem(c * CHUNK, SEQ)

            @plsc.parallel_loop(0, CHUNK, unroll=4)
            def add_body(r):
                p = q + r
                for j in range(VREGS_PER_ROW):
                    sl = pl.ds(j * LANES, LANES)
                    buf[r, sl] = buf[r, sl] + pos_v[p, sl]
            pltpu.sync_copy(buf, out_hbm.at[pl.ds(base + c * CHUNK, CHUNK)])

        gather(0, rows_a, sem_a)

        def pair_body(i, carry):
            c0 = 2 * i
            pltpu.make_async_copy(w_hbm.at[idx_v.at[c0]], rows_a, sem_a).wait()
            gather(c0 + 1, rows_b, sem_b)
            add_and_flush(c0, rows_a)
            pltpu.make_async_copy(w_hbm.at[idx_v.at[c0 + 1]], rows_b, sem_b).wait()
            gather(c0 + 2, rows_a, sem_a)
            add_and_flush(c0 + 1, rows_b)
            return carry

        lax.fori_loop(0, (CHUNKS_PER_W - 1) // 2, pair_body, 0)
        pltpu.make_async_copy(
            w_hbm.at[idx_v.at[CHUNKS_PER_W - 1]], rows_a, sem_a).wait()
        add_and_flush(CHUNKS_PER_W - 1, rows_a)

    return k(ids3, weight, position_embedding)


def kernel(input_ids, weight, position_embedding):
    ids3 = jnp.asarray(input_ids, jnp.int32).reshape(
        NUM_WORKERS, CHUNKS_PER_W, CHUNK)
    pos_ext = jnp.concatenate(
        [position_embedding, position_embedding,
         position_embedding[: POS_EXT - 2 * SEQ]], axis=0)
    out = _sc_embed(ids3, weight, pos_ext)
    return out.reshape(BATCH, SEQ, EMBED)
